# baseline (speedup 1.0000x reference)
import functools

import jax
import jax.numpy as jnp
from jax import lax
from jax.experimental import pallas as pl
from jax.experimental.pallas import tpu as pltpu
from jax.experimental.pallas import tpu_sc as plsc

_B = 1024
_V = 100000
_NC, _NS, _L = 2, 16, 16
_NW = _NC * _NS
_BPW = _B // _NW
_CH = _BPW // _L

_LN2 = 0.6931471805599453


def _neg_log(v):
  bits = lax.bitcast_convert_type(v, jnp.int32)
  e = (lax.shift_right_logical(bits, 23) & 0xFF) - 127
  ef = e.astype(jnp.float32)
  m = lax.bitcast_convert_type((bits & 0x007FFFFF) | 0x3F800000, jnp.float32)
  s = (m - 1.0) / (m + 1.0)
  s2 = s * s
  lnm = 2.0 * s * (1.0 + s2 * (1.0 / 3.0 + s2 * (1.0 / 5.0 + s2 * (1.0 / 7.0 + s2 * (1.0 / 9.0)))))
  ln = ef * _LN2 + lnm
  return jnp.where(v == 0.0, jnp.float32(jnp.inf), -ln)


@functools.partial(
    pl.kernel,
    mesh=plsc.VectorSubcoreMesh(core_axis_name="c", subcore_axis_name="s"),
    out_type=jax.ShapeDtypeStruct((_NW, _L), jnp.float32),
    scratch_types=[
        pltpu.VMEM((_BPW,), jnp.int32),
        pltpu.VMEM((_BPW,), jnp.int32),
        pltpu.VMEM((_BPW,), jnp.float32),
        pltpu.VMEM((_L,), jnp.float32),
        pltpu.SemaphoreType.DMA,
    ],
)
def _ce_gather(flat_hbm, trg_hbm, out_hbm, trg_v, idx_v, vals_v, stage_v, sem):
  wid = lax.axis_index("s") * _NC + lax.axis_index("c")
  base = wid * _BPW
  pltpu.sync_copy(trg_hbm.at[pl.ds(base, _BPW)], trg_v)
  for c in range(_CH):
    rows = (base + c * _L) + lax.iota(jnp.int32, _L)
    idx_v[pl.ds(c * _L, _L)] = rows * _V + trg_v[pl.ds(c * _L, _L)]
  pltpu.async_copy(flat_hbm.at[idx_v], vals_v, sem).wait()
  acc = jnp.zeros((_L,), jnp.float32)
  for c in range(_CH):
    acc = acc + _neg_log(vals_v[pl.ds(c * _L, _L)])
  stage_v[...] = acc
  pltpu.sync_copy(stage_v, out_hbm.at[wid])


def kernel(prd, trg):
  flat = prd.reshape(-1)
  partials = _ce_gather(flat, trg.astype(jnp.int32))
  return jnp.sum(partials)


# speedup vs baseline: 2.3598x; 2.3598x over previous
import functools

import jax
import jax.numpy as jnp
from jax import lax
from jax.experimental import pallas as pl
from jax.experimental.pallas import tpu as pltpu
from jax.experimental.pallas import tpu_sc as plsc

_B = 1024
_V = 100000
_NC, _NS, _L = 2, 16, 16
_NW = _NC * _NS
_BPW = _B // _NW
_CH = _BPW // _L

_LN2 = 0.6931471805599453


def _neg_log(v):
  bits = lax.bitcast_convert_type(v, jnp.int32)
  e = (lax.shift_right_logical(bits, 23) & 0xFF) - 127
  ef = e.astype(jnp.float32)
  m = lax.bitcast_convert_type((bits & 0x007FFFFF) | 0x3F800000, jnp.float32)
  s = (m - 1.0) / (m + 1.0)
  s2 = s * s
  lnm = 2.0 * s * (1.0 + s2 * (1.0 / 3.0 + s2 * (1.0 / 5.0 + s2 * (1.0 / 7.0 + s2 * (1.0 / 9.0)))))
  ln = ef * _LN2 + lnm
  return jnp.where(v == 0.0, jnp.float32(jnp.inf), -ln)


@functools.partial(
    pl.kernel,
    mesh=plsc.VectorSubcoreMesh(core_axis_name="c", subcore_axis_name="s"),
    compiler_params=pltpu.CompilerParams(needs_layout_passes=False),
    out_type=jax.ShapeDtypeStruct((_NW, _L), jnp.float32),
    scratch_types=[
        pltpu.VMEM((_BPW,), jnp.int32),
        pltpu.VMEM((_BPW, 8, 128), jnp.float32),
        pltpu.VMEM((_L,), jnp.float32),
        pltpu.SemaphoreType.DMA,
    ],
)
def _ce_gather(prd_hbm, trg_hbm, out_hbm, trg_v, tiles_v, stage_v, sem):
  wid = lax.axis_index("s") * _NC + lax.axis_index("c")
  base = wid * _BPW
  pltpu.sync_copy(trg_hbm.at[pl.ds(base, _BPW)], trg_v)
  copies = []
  for c in range(_CH):
    tch = trg_v[pl.ds(c * _L, _L)]
    for s in range(_L):
      i = c * _L + s
      t = tch[s]
      colb = pl.multiple_of(lax.shift_left(lax.shift_right_logical(t, 7), 7), 128)
      rb = base + (i // 8) * 8
      copies.append(
          pltpu.async_copy(
              prd_hbm.at[pl.ds(rb, 8), pl.ds(colb, 128)], tiles_v.at[i], sem
          )
      )
  for cp in copies:
    cp.wait()
  acc = jnp.zeros((_L,), jnp.float32)
  for c in range(_CH):
    tch = trg_v[pl.ds(c * _L, _L)]
    samp = lax.iota(jnp.int32, _L) + c * _L
    subrow = samp & 7
    lane = tch & 127
    vals = plsc.load_gather(tiles_v, [samp, subrow, lane])
    acc = acc + _neg_log(vals)
  stage_v[...] = acc
  pltpu.sync_copy(stage_v, out_hbm.at[wid])


def kernel(prd, trg):
  partials = _ce_gather(prd, trg.astype(jnp.int32))
  return jnp.sum(partials)
